# baseline (speedup 1.0000x reference)
import functools

import jax
import jax.numpy as jnp
from jax import lax
from jax.experimental import pallas as pl
from jax.experimental.pallas import tpu as pltpu
from jax.experimental.pallas import tpu_sc as plsc


def _sum64_sublanes(sq):
    a = sq[0:8]
    for g in range(1, 8):
        a = a + sq[8 * g:8 * g + 8]
    b = a[0:4] + a[4:8]
    c = b[0:2] + b[2:4]
    return c[0:1] + c[1:2]


def _rowsum64(r):
    a = r[:, 0:8]
    for g in range(1, 8):
        a = a + r[:, 8 * g:8 * g + 8]
    b = a[:, 0:4] + a[:, 4:8]
    c = b[:, 0:2] + b[:, 2:4]
    return c[:, 0:1] + c[:, 1:2]


def _dist_argmax_body(x_ref, cb_ref, idx_ref, c2_ref):
    xt = x_ref[...].T
    cb = cb_ref[...]
    k = cb.shape[0]

    @pl.when(pl.program_id(0) == 0)
    def _():
        c2_ref[...] = jnp.broadcast_to(_rowsum64(cb * cb), (k, 128))

    x2 = _sum64_sublanes(xt * xt)
    c2 = c2_ref[...][:, 0:1]
    mm = lax.dot_general(cb, xt, (((1,), (0,)), ((), ())),
                         preferred_element_type=jnp.float32)
    d2 = x2 + c2 - 2.0 * mm
    dist = jnp.sqrt(jnp.maximum(d2, 0.0))
    m = jnp.max(dist, axis=0, keepdims=True)
    iota = lax.broadcasted_iota(jnp.int32, dist.shape, 0)
    cand_idx = jnp.where(dist == m, iota, k)
    idx_ref[...] = jnp.min(cand_idx, axis=0).astype(jnp.int32)


def _dist_argmax(x, codebook, block_n=1024):
    n, d = x.shape
    k = codebook.shape[0]
    return pl.pallas_call(
        _dist_argmax_body,
        grid=(n // block_n,),
        in_specs=[
            pl.BlockSpec((block_n, d), lambda i: (i, 0)),
            pl.BlockSpec((k, d), lambda i: (0, 0)),
        ],
        out_specs=pl.BlockSpec((block_n,), lambda i: (i,)),
        out_shape=jax.ShapeDtypeStruct((n,), jnp.int32),
        scratch_shapes=[pltpu.VMEM((k, 128), jnp.float32)],
    )(x, codebook)


@functools.lru_cache(maxsize=None)
def _make_sc_gather(k, d, n):
    info = plsc.get_sparse_core_info()
    nw = info.num_cores * info.num_subcores
    assert d % info.num_lanes == 0 and n % (8 * nw) == 0
    b_per_w = n // nw
    chunk = 384
    n_chunks = b_per_w // chunk
    assert b_per_w % chunk == 0
    mesh = plsc.VectorSubcoreMesh(core_axis_name="c", subcore_axis_name="s")

    @functools.partial(
        pl.kernel, mesh=mesh,
        out_type=jax.ShapeDtypeStruct((n, 128), jnp.float32),
        scratch_types=[
            pltpu.VMEM((b_per_w,), jnp.int32),
            pltpu.VMEM((chunk, 128), jnp.float32),
            pltpu.VMEM((chunk, 128), jnp.float32),
            pltpu.SemaphoreType.DMA,
            pltpu.SemaphoreType.DMA,
            pltpu.SemaphoreType.DMA,
            pltpu.SemaphoreType.DMA,
        ],
    )
    def gather(table_hbm, idx_hbm, out_hbm, idx_v, rows_a, rows_b,
               gsem_a, gsem_b, csem_a, csem_b):
        wid = lax.axis_index("s") * info.num_cores + lax.axis_index("c")
        base = wid * b_per_w
        bufs = (rows_a, rows_b)
        gsems = (gsem_a, gsem_b)
        csems = (csem_a, csem_b)
        pltpu.sync_copy(idx_hbm.at[pl.ds(base, b_per_w)], idx_v)

        def start_gather(j):
            b = j & 1
            return pltpu.async_copy(
                table_hbm.at[idx_v.at[pl.ds(j * chunk, chunk)]],
                bufs[b], gsems[b])

        g = start_gather(0)
        copies = [None, None]
        for j in range(n_chunks):
            b = j & 1
            g.wait()
            if j + 1 < n_chunks:
                b2 = (j + 1) & 1
                if copies[b2] is not None:
                    copies[b2].wait()
                g = start_gather(j + 1)
            copies[b] = pltpu.async_copy(
                bufs[b], out_hbm.at[pl.ds(base + j * chunk, chunk)],
                csems[b])
        for c in copies:
            if c is not None:
                c.wait()

    return gather


def kernel(inputs, codebook):
    shape = inputs.shape
    x = inputs.reshape(-1, shape[-1]) if inputs.ndim > 2 else inputs
    n, d = x.shape
    k = codebook.shape[0]
    idx = _dist_argmax(x, codebook)
    cb_pad = jnp.pad(codebook, ((0, 0), (0, 128 - d)))
    quantized = _make_sc_gather(k, d, n)(cb_pad, idx)[:, :d]
    return quantized.reshape(shape)


# speedup vs baseline: 1.1122x; 1.1122x over previous
import functools

import jax
import jax.numpy as jnp
from jax import lax
from jax.experimental import pallas as pl
from jax.experimental.pallas import tpu as pltpu
from jax.experimental.pallas import tpu_sc as plsc


def _sum64_sublanes(sq):
    a = sq[0:8]
    for g in range(1, 8):
        a = a + sq[8 * g:8 * g + 8]
    b = a[0:4] + a[4:8]
    c = b[0:2] + b[2:4]
    return c[0:1] + c[1:2]


def _rowsum64(r):
    a = r[:, 0:8]
    for g in range(1, 8):
        a = a + r[:, 8 * g:8 * g + 8]
    b = a[:, 0:4] + a[:, 4:8]
    c = b[:, 0:2] + b[:, 2:4]
    return c[:, 0:1] + c[:, 1:2]


def _c2_body(cb_ref, c2_ref):
    cb = cb_ref[...]
    c2_ref[...] = jnp.broadcast_to(_rowsum64(cb * cb), (cb.shape[0], 128))


def _dist_argmax_body(x_ref, cb_ref, c2_ref, idx_ref):
    xt = x_ref[...].T
    cb = cb_ref[...]
    k = cb.shape[0]
    x2 = _sum64_sublanes(xt * xt)
    c2 = c2_ref[...][:, 0:1]
    mm = lax.dot_general(cb, xt, (((1,), (0,)), ((), ())),
                         preferred_element_type=jnp.float32)
    d2 = x2 + c2 - 2.0 * mm
    xc = jnp.maximum(d2, 0.0)
    dist = jnp.where(xc == 0.0, 0.0, xc * lax.rsqrt(xc))
    m = jnp.max(dist, axis=0, keepdims=True)
    iota = lax.broadcasted_iota(jnp.int32, dist.shape, 0)
    cand_idx = jnp.where(dist == m, iota, k)
    idx_ref[...] = jnp.min(cand_idx, axis=0).astype(jnp.int32)


def _dist_argmax(x, codebook, block_n=1024):
    n, d = x.shape
    k = codebook.shape[0]
    c2 = pl.pallas_call(
        _c2_body,
        in_specs=[pl.BlockSpec((k, d), lambda: (0, 0))],
        out_specs=pl.BlockSpec((k, 128), lambda: (0, 0)),
        out_shape=jax.ShapeDtypeStruct((k, 128), jnp.float32),
    )(codebook)
    return pl.pallas_call(
        _dist_argmax_body,
        grid=(n // block_n,),
        in_specs=[
            pl.BlockSpec((block_n, d), lambda i: (i, 0)),
            pl.BlockSpec((k, d), lambda i: (0, 0)),
            pl.BlockSpec((k, 128), lambda i: (0, 0)),
        ],
        out_specs=pl.BlockSpec((block_n,), lambda i: (i,)),
        out_shape=jax.ShapeDtypeStruct((n,), jnp.int32),
    )(x, codebook, c2)


@functools.lru_cache(maxsize=None)
def _make_sc_gather(k, d, n):
    info = plsc.get_sparse_core_info()
    nw = info.num_cores * info.num_subcores
    assert d % info.num_lanes == 0 and n % (8 * nw) == 0
    b_per_w = n // nw
    chunk = 384
    n_chunks = b_per_w // chunk
    assert b_per_w % chunk == 0
    mesh = plsc.VectorSubcoreMesh(core_axis_name="c", subcore_axis_name="s")

    @functools.partial(
        pl.kernel, mesh=mesh,
        out_type=jax.ShapeDtypeStruct((n, 128), jnp.float32),
        scratch_types=[
            pltpu.VMEM((b_per_w,), jnp.int32),
            pltpu.VMEM((chunk, 128), jnp.float32),
            pltpu.VMEM((chunk, 128), jnp.float32),
            pltpu.SemaphoreType.DMA,
            pltpu.SemaphoreType.DMA,
            pltpu.SemaphoreType.DMA,
            pltpu.SemaphoreType.DMA,
        ],
    )
    def gather(table_hbm, idx_hbm, out_hbm, idx_v, rows_a, rows_b,
               gsem_a, gsem_b, csem_a, csem_b):
        wid = lax.axis_index("s") * info.num_cores + lax.axis_index("c")
        base = wid * b_per_w
        bufs = (rows_a, rows_b)
        gsems = (gsem_a, gsem_b)
        csems = (csem_a, csem_b)
        pltpu.sync_copy(idx_hbm.at[pl.ds(base, b_per_w)], idx_v)

        def start_gather(j):
            b = j & 1
            return pltpu.async_copy(
                table_hbm.at[idx_v.at[pl.ds(j * chunk, chunk)]],
                bufs[b], gsems[b])

        g = start_gather(0)
        copies = [None, None]
        for j in range(n_chunks):
            b = j & 1
            g.wait()
            if j + 1 < n_chunks:
                b2 = (j + 1) & 1
                if copies[b2] is not None:
                    copies[b2].wait()
                g = start_gather(j + 1)
            copies[b] = pltpu.async_copy(
                bufs[b], out_hbm.at[pl.ds(base + j * chunk, chunk)],
                csems[b])
        for c in copies:
            if c is not None:
                c.wait()

    return gather


def kernel(inputs, codebook):
    shape = inputs.shape
    x = inputs.reshape(-1, shape[-1]) if inputs.ndim > 2 else inputs
    n, d = x.shape
    k = codebook.shape[0]
    idx = _dist_argmax(x, codebook)
    cb_pad = jnp.pad(codebook, ((0, 0), (0, 128 - d)))
    quantized = _make_sc_gather(k, d, n)(cb_pad, idx)[:, :d]
    return quantized.reshape(shape)
